# baseline (speedup 1.0000x reference)
import jax
import jax.numpy as jnp
from jax import lax
from jax.experimental import pallas as pl
from jax.experimental.pallas import tpu as pltpu
from jax.experimental.pallas import tpu_sc as plsc

N = 100000
E = 3200000
LANES = 128
R = E // LANES
K = 25
CHUNKS = R // K
NC, NS = 2, 16
NW = NC * NS
NPAD = 100352
STRIPE = NPAD // NS
BLK = 2048
NB = NPAD // BLK


def _deg_body(e, ones_hbm, z1_hbm, out, deg, ones_v, idx_v, stage1, sem):
    cid = lax.axis_index("c")
    sid = lax.axis_index("s")
    wid = cid * NS + sid
    off = sid * STRIPE
    pltpu.sync_copy(ones_hbm, ones_v)
    pltpu.sync_copy(z1_hbm, stage1)
    pltpu.sync_copy(stage1, deg.at[pl.ds(off, STRIPE)])
    plsc.subcore_barrier()
    n_chunks = (CHUNKS - wid + NW - 1) // NW

    def body(i, carry):
        row0 = (wid + NW * i) * K
        pltpu.sync_copy(e.at[1, pl.ds(row0, K)], idx_v)
        descs = [
            pltpu.async_copy(ones_v, deg.at[idx_v.at[j]], sem, add=True)
            for j in range(K)
        ]
        for d in descs:
            d.wait()
        return carry

    lax.fori_loop(0, n_chunks, body, 0)
    plsc.subcore_barrier()
    pltpu.sync_copy(deg.at[pl.ds(off, STRIPE)], stage1)
    pltpu.sync_copy(stage1, out.at[pl.ds(cid * NPAD + off, STRIPE)])


def _msg_body(e, g4, dv, z4_hbm, z1_hbm, ao, so,
              agg, s, idx2, rows, dvals, stage4, stage1, sem):
    cid = lax.axis_index("c")
    sid = lax.axis_index("s")
    wid = cid * NS + sid
    off = sid * STRIPE
    pltpu.sync_copy(z4_hbm, stage4)
    pltpu.sync_copy(stage4, agg.at[pl.ds(off, STRIPE), :])
    pltpu.sync_copy(z1_hbm, stage1)
    pltpu.sync_copy(stage1, s.at[pl.ds(off, STRIPE)])
    plsc.subcore_barrier()
    n_chunks = (CHUNKS - wid + NW - 1) // NW

    def body(i, carry):
        row0 = (wid + NW * i) * K
        pltpu.sync_copy(e.at[:, pl.ds(row0, K)], idx2)
        descs = [
            pltpu.async_copy(g4.at[idx2.at[0, j]], rows.at[j], sem)
            for j in range(K)
        ] + [
            pltpu.async_copy(dv.at[idx2.at[1, j]], dvals.at[j], sem)
            for j in range(K)
        ]
        for d in descs:
            d.wait()
        descs = [
            pltpu.async_copy(rows.at[j], agg.at[idx2.at[1, j]], sem, add=True)
            for j in range(K)
        ] + [
            pltpu.async_copy(dvals.at[j], s.at[idx2.at[0, j]], sem, add=True)
            for j in range(K)
        ]
        for d in descs:
            d.wait()
        return carry

    lax.fori_loop(0, n_chunks, body, 0)
    plsc.subcore_barrier()
    pltpu.sync_copy(agg.at[pl.ds(off, STRIPE), :], stage4)
    pltpu.sync_copy(stage4, ao.at[pl.ds(cid * NPAD + off, STRIPE), :])
    pltpu.sync_copy(s.at[pl.ds(off, STRIPE)], stage1)
    pltpu.sync_copy(stage1, so.at[pl.ds(cid * NPAD + off, STRIPE)])


def _prep_body(dp, x, dv, g4):
    i = pl.program_id(0)
    rowid = lax.broadcasted_iota(jnp.int32, (1, BLK), 1) + i * BLK
    mask = rowid < N
    dsum = dp[0:1, :] + dp[1:2, :] + 1.0
    dinv = jnp.where(mask, lax.rsqrt(dsum), 0.0)
    dv[...] = jnp.reshape(dinv, (BLK,))
    dcol = jnp.transpose(dinv)
    g4[...] = jnp.where(dcol > 0.0, x[...] * dcol, 0.0)


def _acc_body(ap, sp, dv, g4, w1, b1, out_ref):
    i = pl.program_id(0)

    @pl.when(i == 0)
    def _():
        out_ref[...] = jnp.zeros_like(out_ref)

    z4 = ap[0] + ap[1] + g4[...]
    zw = jnp.dot(z4, w1[...], preferred_element_type=jnp.float32)
    drow = jnp.reshape(dv[...], (1, BLK))
    dcol = jnp.transpose(drow)
    h = jax.nn.relu(zw * dcol + b1[...])
    srow = sp[0:1, :] + sp[1:2, :]
    crow = drow * (srow + drow)
    out_ref[...] += jnp.dot(crow, h, preferred_element_type=jnp.float32)


def _head_body(m_t, m_e, m_p, w2_t, w2_e, w2_p, b2_t, b2_e, b2_p,
               wfc, bfc, out_ref):
    ms = []
    for m, w2, b2 in ((m_t, w2_t, b2_t), (m_e, w2_e, b2_e), (m_p, w2_p, b2_p)):
        o = jnp.dot(m[...], w2[...], preferred_element_type=jnp.float32)
        ms.append(o / float(N) + b2[...])
    comb = jnp.concatenate(ms, axis=1)
    o = jnp.dot(comb, wfc[...], preferred_element_type=jnp.float32)
    out_ref[...] = jax.nn.sigmoid(o + bfc[...])


def kernel(target_x, target_edge_index, e3_ligase_x, e3_ligase_edge_index,
           protac_x, protac_edge_index, W1t, b1t, W2t, b2t, W1e, b1e, W2e, b2e,
           W1p, b1p, W2p, b2p, Wfc, bfc):
    f32 = jnp.float32
    ones_hbm = jnp.ones((LANES,), f32)
    z1_hbm = jnp.zeros((STRIPE,), f32)
    z4_hbm = jnp.zeros((STRIPE, 4), f32)

    mesh = plsc.VectorSubcoreMesh(
        core_axis_name="c", subcore_axis_name="s",
        num_cores=NC, num_subcores=NS)

    deg_call = pl.kernel(
        _deg_body,
        compiler_params=pltpu.CompilerParams(use_tc_tiling_on_sc=False),
        out_type=jax.ShapeDtypeStruct((NC * NPAD,), f32),
        mesh=mesh,
        scratch_types=[
            pltpu.VMEM_SHARED((NPAD,), f32),
            pltpu.VMEM((LANES,), f32),
            pltpu.VMEM((K, LANES), jnp.int32),
            pltpu.VMEM((STRIPE,), f32),
            pltpu.SemaphoreType.DMA,
        ],
    )

    prep_call = pl.pallas_call(
        _prep_body,
        grid=(NB,),
        in_specs=[pl.BlockSpec((NC, BLK), lambda i: (0, i)),
                  pl.BlockSpec((BLK, 4), lambda i: (i, 0))],
        out_specs=[pl.BlockSpec((BLK,), lambda i: (i,)),
                   pl.BlockSpec((BLK, 4), lambda i: (i, 0))],
        out_shape=[jax.ShapeDtypeStruct((NPAD,), f32),
                   jax.ShapeDtypeStruct((NPAD, 4), f32)],
    )

    msg_call = pl.kernel(
        _msg_body,
        compiler_params=pltpu.CompilerParams(use_tc_tiling_on_sc=False),
        out_type=[jax.ShapeDtypeStruct((NC * NPAD, 4), f32),
                  jax.ShapeDtypeStruct((NC * NPAD,), f32)],
        mesh=mesh,
        scratch_types=[
            pltpu.VMEM_SHARED((NPAD, 4), f32),
            pltpu.VMEM_SHARED((NPAD,), f32),
            pltpu.VMEM((2, K, LANES), jnp.int32),
            pltpu.VMEM((K, LANES, 4), f32),
            pltpu.VMEM((K, LANES), f32),
            pltpu.VMEM((STRIPE, 4), f32),
            pltpu.VMEM((STRIPE,), f32),
            pltpu.SemaphoreType.DMA,
        ],
    )

    full = lambda s: pl.BlockSpec(s, lambda i: tuple(0 for _ in s))
    acc_call = pl.pallas_call(
        _acc_body,
        grid=(NB,),
        in_specs=[pl.BlockSpec((NC, BLK, 4), lambda i: (0, i, 0)),
                  pl.BlockSpec((NC, BLK), lambda i: (0, i)),
                  pl.BlockSpec((BLK,), lambda i: (i,)),
                  pl.BlockSpec((BLK, 4), lambda i: (i, 0)),
                  full((4, 64)), full((1, 64))],
        out_specs=pl.BlockSpec((1, 64), lambda i: (0, 0)),
        out_shape=jax.ShapeDtypeStruct((1, 64), f32),
    )

    graphs = (
        (target_x, target_edge_index, W1t, b1t),
        (e3_ligase_x, e3_ligase_edge_index, W1e, b1e),
        (protac_x, protac_edge_index, W1p, b1p),
    )
    es = [jnp.reshape(g[1].astype(jnp.int32), (2, R, LANES)) for g in graphs]
    degs = [deg_call(e, ones_hbm, z1_hbm).reshape(NC, NPAD) for e in es]
    preps = [prep_call(degp, g[0]) for degp, g in zip(degs, graphs)]
    msgs = [msg_call(e, g4, dv, z4_hbm, z1_hbm)
            for e, (dv, g4) in zip(es, preps)]
    msums = [acc_call(ao.reshape(NC, NPAD, 4), so.reshape(NC, NPAD),
                      dv, g4, g[2], g[3].reshape(1, 64))
             for (ao, so), (dv, g4), g in zip(msgs, preps, graphs)]

    full0 = lambda s: pl.BlockSpec(s, lambda: tuple(0 for _ in s))
    out = pl.pallas_call(
        _head_body,
        in_specs=[full0((1, 64))] * 3 + [full0((64, 32))] * 3
                 + [full0((1, 32))] * 3 + [full0((96, 1)), full0((1, 1))],
        out_specs=full0((1, 1)),
        out_shape=jax.ShapeDtypeStruct((1, 1), f32),
    )(*msums, W2t, W2e, W2p, b2t.reshape(1, 32), b2e.reshape(1, 32),
      b2p.reshape(1, 32), Wfc, bfc.reshape(1, 1))
    return out.reshape(1)
